# baseline (speedup 1.0000x reference)
import functools

import numpy as np
import jax
import jax.numpy as jnp
from jax.experimental import pallas as pl
from jax.experimental.pallas import tpu as pltpu

_N = 96
_K = 5
_GB = 16
_ROWS = _GB * _N
_G = 256
_F = 256
_H = 128
_HT = 256
_NH = 8
_DH = 32
_B = 8
_T = 32
_NCHUNK = _G // _GB


def _band_adjacency() -> np.ndarray:
    src, dst = [], []
    for i in range(_N):
        for j in range(max(0, i - _K), min(_N, i + _K + 1)):
            if i != j:
                src.append(i)
                dst.append(j)
    src = np.asarray(src)
    dst = np.asarray(dst)
    deg = np.zeros(_N, np.float64)
    np.add.at(deg, dst, 1.0)
    deg += 1.0
    dinv = 1.0 / np.sqrt(deg)
    A = np.zeros((_N, _N), np.float64)
    A[dst, src] = dinv[src] * dinv[dst]
    A[np.arange(_N), np.arange(_N)] = dinv * dinv
    return A.astype(np.float32)


_DOT = functools.partial(jnp.dot, preferred_element_type=jnp.float32,
                         precision=jax.lax.Precision.DEFAULT)


def _ln(x, g, b):
    m = jnp.mean(x, axis=-1, keepdims=True)
    v = jnp.mean((x - m) ** 2, axis=-1, keepdims=True)
    return (x - m) * jax.lax.rsqrt(v + 1e-5) * g + b


def _fused_kernel(x_ref, w0_ref, b0_ref, w1_ref, b1_ref, w2_ref, b2_ref,
                  a_ref, nf_ref, tinw_ref, tinb_ref,
                  wq0_ref, wk0_ref, wv0_ref, wo0_ref, f10_ref, f1b0_ref,
                  f20_ref, f2b0_ref, g10_ref, b10_ref, g20_ref, b20_ref,
                  wq1_ref, wk1_ref, wv1_ref, wo1_ref, f11_ref, f1b1_ref,
                  f21_ref, f2b1_ref, g11_ref, b11_ref, g21_ref, b21_ref,
                  cw0_ref, cb0_ref, cg0_ref, cbt0_ref,
                  cw1_ref, cb1_ref, cg1_ref, cbt1_ref,
                  cw2_ref, cb2_ref, out_ref, ge_scr):
    i = pl.program_id(0)

    @pl.when(i < _NCHUNK)
    def _spatial():
        ahat = a_ref[:]

        def band(h):
            htile = jnp.concatenate(
                [h[g * _N:(g + 1) * _N, :] for g in range(_GB)], axis=1)
            otile = _DOT(ahat, htile)
            return jnp.concatenate(
                [otile[:, g * _H:(g + 1) * _H] for g in range(_GB)], axis=0)

        h = _DOT(x_ref[:].astype(jnp.bfloat16),
                 w0_ref[:].astype(jnp.bfloat16))
        h = jnp.maximum(band(h) + b0_ref[:], 0.0)
        h = _DOT(h, w1_ref[:])
        h = jnp.maximum(band(h) + b1_ref[:], 0.0)
        h = _DOT(h, w2_ref[:])
        h = band(h) + b2_ref[:]
        means, maxs = [], []
        for g in range(_GB):
            seg = h[g * _N:(g + 1) * _N, :]
            means.append(jnp.mean(seg, axis=0, keepdims=True))
            maxs.append(jnp.max(seg, axis=0, keepdims=True))
        ge_chunk = jnp.concatenate(
            [jnp.concatenate(means, axis=0), jnp.concatenate(maxs, axis=0)],
            axis=1)
        ge_scr[pl.ds(i * _GB, _GB), :] = ge_chunk

    @pl.when(i == _NCHUNK)
    def _temporal():
        hT = _DOT(ge_scr[:], tinw_ref[:]) + tinb_ref[:]
        nf = nf_ref[:]
        t_iota = jax.lax.broadcasted_iota(jnp.int32, (_B, _T), 1)
        pad = t_iota >= nf
        scale = jnp.float32(1.0 / np.sqrt(_DH))
        BT = _B * _T

        r_blk = jax.lax.broadcasted_iota(jnp.int32, (BT, BT), 0) // _DH
        c_blk = jax.lax.broadcasted_iota(jnp.int32, (BT, BT), 1) // _DH
        M = (r_blk == c_blk).astype(jnp.float32)

        mrows = []
        for bi in range(_B):
            row = jnp.where(pad[bi:bi + 1, :], jnp.float32(-1e9), 0.0)
            rowt = jnp.concatenate([row] * _NH, axis=1)
            mrows.append(jnp.broadcast_to(rowt, (_T, BT)))
        addend = jnp.concatenate(mrows, axis=0)

        layers = (
            (wq0_ref, wk0_ref, wv0_ref, wo0_ref, f10_ref, f1b0_ref, f20_ref,
             f2b0_ref, g10_ref, b10_ref, g20_ref, b20_ref),
            (wq1_ref, wk1_ref, wv1_ref, wo1_ref, f11_ref, f1b1_ref, f21_ref,
             f2b1_ref, g11_ref, b11_ref, g21_ref, b21_ref),
        )
        hT_cur = hT
        for (wq, wk, wv, wo, f1, f1b, f2, f2b, g1, b1, g2, b2) in layers:
            q = _DOT(hT_cur, wq[:])
            k = _DOT(hT_cur, wk[:])
            v = _DOT(hT_cur, wv[:])
            kT = k.T
            srows = []
            for bi in range(_B):
                kTb = kT[:, bi * _T:(bi + 1) * _T]
                Kp = jnp.concatenate([kTb] * _NH, axis=1) * M
                srows.append(_DOT(q[bi * _T:(bi + 1) * _T, :], Kp))
            S2 = jnp.concatenate(srows, axis=0) * scale + addend
            mx = jnp.max(S2)
            e = jnp.exp(S2 - mx)
            denom = _DOT(e, M)
            A2 = e / denom
            orows = []
            for bi in range(_B):
                vb = v[bi * _T:(bi + 1) * _T, :]
                Vp = jnp.concatenate([vb] * _NH, axis=0) * M
                orows.append(_DOT(A2[bi * _T:(bi + 1) * _T, :], Vp))
            O = jnp.concatenate(orows, axis=0)
            hT_cur = _ln(hT_cur + _DOT(O, wo[:]), g1[:], b1[:])
            f = _DOT(jnp.maximum(_DOT(hT_cur, f1[:]) + f1b[:], 0.0),
                     f2[:]) + f2b[:]
            hT_cur = _ln(hT_cur + f, g2[:], b2[:])

        valid = jnp.where(pad, 0.0, 1.0).astype(jnp.float32)
        pooled_rows = []
        for bi in range(_B):
            vb = valid[bi:bi + 1, :]
            pooled_rows.append(_DOT(vb, hT_cur[bi * _T:(bi + 1) * _T, :]))
        pooled = jnp.concatenate(pooled_rows, axis=0) / nf.astype(jnp.float32)

        z = jnp.maximum(_ln(_DOT(pooled, cw0_ref[:]) + cb0_ref[:],
                            cg0_ref[:], cbt0_ref[:]), 0.0)
        z = jnp.maximum(_ln(_DOT(z, cw1_ref[:]) + cb1_ref[:],
                            cg1_ref[:], cbt1_ref[:]), 0.0)
        out_ref[:] = _DOT(z, cw2_ref[:]) + cb2_ref[:]


def kernel(x_temporal, num_frames, gcn_W0, gcn_b0, gcn_W1, gcn_b1, gcn_W2,
           gcn_b2, tin_W, tin_b, Wq0, Wk0, Wv0, Wo0, ff1_0, ff1b_0, ff2_0,
           ff2b_0, ln1g_0, ln1b_0, ln2g_0, ln2b_0, Wq1, Wk1, Wv1, Wo1, ff1_1,
           ff1b_1, ff2_1, ff2b_1, ln1g_1, ln1b_1, ln2g_1, ln2b_1, cls_W0,
           cls_b0, cls_ln0g, cls_ln0b, cls_W1, cls_b1, cls_ln1g, cls_ln1b,
           cls_W2, cls_b2):
    Bb, Tt, Nn, Ff = x_temporal.shape
    xf = x_temporal.reshape(Bb * Tt * Nn, Ff)
    A = jnp.asarray(_band_adjacency())
    r2 = lambda a: a.reshape(1, -1)

    const = lambda i: (0, 0)
    xmap = lambda i: (jnp.minimum(i, _NCHUNK - 1), 0)
    spatial_args = (xf, gcn_W0, r2(gcn_b0), gcn_W1, r2(gcn_b1), gcn_W2,
                    r2(gcn_b2), A)
    temporal_args = (
        num_frames, tin_W, r2(tin_b),
        Wq0, Wk0, Wv0, Wo0, ff1_0, r2(ff1b_0), ff2_0, r2(ff2b_0),
        r2(ln1g_0), r2(ln1b_0), r2(ln2g_0), r2(ln2b_0),
        Wq1, Wk1, Wv1, Wo1, ff1_1, r2(ff1b_1), ff2_1, r2(ff2b_1),
        r2(ln1g_1), r2(ln1b_1), r2(ln2g_1), r2(ln2b_1),
        cls_W0, r2(cls_b0), r2(cls_ln0g), r2(cls_ln0b),
        cls_W1, r2(cls_b1), r2(cls_ln1g), r2(cls_ln1b),
        cls_W2, r2(cls_b2))
    out = pl.pallas_call(
        _fused_kernel,
        grid=(_NCHUNK + 1,),
        in_specs=[
            pl.BlockSpec((_ROWS, _F), xmap),
            pl.BlockSpec((_F, _H), const),
            pl.BlockSpec((1, _H), const),
            pl.BlockSpec((_H, _H), const),
            pl.BlockSpec((1, _H), const),
            pl.BlockSpec((_H, _H), const),
            pl.BlockSpec((1, _H), const),
            pl.BlockSpec((_N, _N), const),
        ] + [pl.BlockSpec(t.shape, const) for t in temporal_args],
        out_specs=pl.BlockSpec((_B, 8), const),
        out_shape=jax.ShapeDtypeStruct((_B, 8), jnp.float32),
        scratch_shapes=[pltpu.VMEM((_G, 2 * _H), jnp.float32)],
        compiler_params=pltpu.CompilerParams(
            dimension_semantics=("arbitrary",)),
    )(*spatial_args, *temporal_args)
    return out


# speedup vs baseline: 1.1334x; 1.1334x over previous
import functools

import numpy as np
import jax
import jax.numpy as jnp
from jax.experimental import pallas as pl
from jax.experimental.pallas import tpu as pltpu

_N = 96
_K = 5
_GB = 32
_ROWS = _GB * _N
_G = 256
_F = 256
_H = 128
_HT = 256
_NH = 8
_DH = 32
_B = 8
_T = 32
_NCHUNK = _G // _GB


def _band_adjacency() -> np.ndarray:
    src, dst = [], []
    for i in range(_N):
        for j in range(max(0, i - _K), min(_N, i + _K + 1)):
            if i != j:
                src.append(i)
                dst.append(j)
    src = np.asarray(src)
    dst = np.asarray(dst)
    deg = np.zeros(_N, np.float64)
    np.add.at(deg, dst, 1.0)
    deg += 1.0
    dinv = 1.0 / np.sqrt(deg)
    A = np.zeros((_N, _N), np.float64)
    A[dst, src] = dinv[src] * dinv[dst]
    A[np.arange(_N), np.arange(_N)] = dinv * dinv
    return A.astype(np.float32)


_DOT = functools.partial(jnp.dot, preferred_element_type=jnp.float32,
                         precision=jax.lax.Precision.DEFAULT)


def _ln(x, g, b):
    m = jnp.mean(x, axis=-1, keepdims=True)
    v = jnp.mean((x - m) ** 2, axis=-1, keepdims=True)
    return (x - m) * jax.lax.rsqrt(v + 1e-5) * g + b


def _fused_kernel(x_ref, w0_ref, b0_ref, w1_ref, b1_ref, w2_ref, b2_ref,
                  a_ref, nf_ref, tinw_ref, tinb_ref,
                  wq0_ref, wk0_ref, wv0_ref, wo0_ref, f10_ref, f1b0_ref,
                  f20_ref, f2b0_ref, g10_ref, b10_ref, g20_ref, b20_ref,
                  wq1_ref, wk1_ref, wv1_ref, wo1_ref, f11_ref, f1b1_ref,
                  f21_ref, f2b1_ref, g11_ref, b11_ref, g21_ref, b21_ref,
                  cw0_ref, cb0_ref, cg0_ref, cbt0_ref,
                  cw1_ref, cb1_ref, cg1_ref, cbt1_ref,
                  cw2_ref, cb2_ref, out_ref, ge_scr):
    i = pl.program_id(0)

    @pl.when(i < _NCHUNK)
    def _spatial():
        ahat = a_ref[:]

        def band(h):
            htile = jnp.concatenate(
                [h[g * _N:(g + 1) * _N, :] for g in range(_GB)], axis=1)
            otile = _DOT(ahat, htile)
            return jnp.concatenate(
                [otile[:, g * _H:(g + 1) * _H] for g in range(_GB)], axis=0)

        h = _DOT(x_ref[:], w0_ref[:])
        h = jnp.maximum(band(h) + b0_ref[:], 0.0)
        h = _DOT(h, w1_ref[:])
        h = jnp.maximum(band(h) + b1_ref[:], 0.0)
        h = _DOT(h, w2_ref[:])
        h = band(h) + b2_ref[:]
        means, maxs = [], []
        for g in range(_GB):
            seg = h[g * _N:(g + 1) * _N, :]
            means.append(jnp.mean(seg, axis=0, keepdims=True))
            maxs.append(jnp.max(seg, axis=0, keepdims=True))
        ge_chunk = jnp.concatenate(
            [jnp.concatenate(means, axis=0), jnp.concatenate(maxs, axis=0)],
            axis=1)
        ge_scr[pl.ds(i * _GB, _GB), :] = ge_chunk

    @pl.when(i == _NCHUNK)
    def _temporal():
        hT = _DOT(ge_scr[:], tinw_ref[:]) + tinb_ref[:]
        nf = nf_ref[:]
        t_iota = jax.lax.broadcasted_iota(jnp.int32, (_B, _T), 1)
        pad = t_iota >= nf
        scale = jnp.float32(1.0 / np.sqrt(_DH))
        BT = _B * _T

        r_blk = jax.lax.broadcasted_iota(jnp.int32, (BT, BT), 0) // _DH
        c_blk = jax.lax.broadcasted_iota(jnp.int32, (BT, BT), 1) // _DH
        M = (r_blk == c_blk).astype(jnp.float32)

        mrows = []
        for bi in range(_B):
            row = jnp.where(pad[bi:bi + 1, :], jnp.float32(-1e9), 0.0)
            rowt = jnp.concatenate([row] * _NH, axis=1)
            mrows.append(jnp.broadcast_to(rowt, (_T, BT)))
        addend = jnp.concatenate(mrows, axis=0)

        layers = (
            (wq0_ref, wk0_ref, wv0_ref, wo0_ref, f10_ref, f1b0_ref, f20_ref,
             f2b0_ref, g10_ref, b10_ref, g20_ref, b20_ref),
            (wq1_ref, wk1_ref, wv1_ref, wo1_ref, f11_ref, f1b1_ref, f21_ref,
             f2b1_ref, g11_ref, b11_ref, g21_ref, b21_ref),
        )
        hT_cur = hT
        for (wq, wk, wv, wo, f1, f1b, f2, f2b, g1, b1, g2, b2) in layers:
            q = _DOT(hT_cur, wq[:])
            k = _DOT(hT_cur, wk[:])
            v = _DOT(hT_cur, wv[:])
            kT = k.T
            srows = []
            for bi in range(_B):
                kTb = kT[:, bi * _T:(bi + 1) * _T]
                Kp = jnp.concatenate([kTb] * _NH, axis=1) * M
                srows.append(_DOT(q[bi * _T:(bi + 1) * _T, :], Kp))
            S2 = jnp.concatenate(srows, axis=0) * scale + addend
            mx = jnp.max(S2)
            e = jnp.exp(S2 - mx)
            denom = _DOT(e, M)
            A2 = e / denom
            orows = []
            for bi in range(_B):
                vb = v[bi * _T:(bi + 1) * _T, :]
                Vp = jnp.concatenate([vb] * _NH, axis=0) * M
                orows.append(_DOT(A2[bi * _T:(bi + 1) * _T, :], Vp))
            O = jnp.concatenate(orows, axis=0)
            hT_cur = _ln(hT_cur + _DOT(O, wo[:]), g1[:], b1[:])
            f = _DOT(jnp.maximum(_DOT(hT_cur, f1[:]) + f1b[:], 0.0),
                     f2[:]) + f2b[:]
            hT_cur = _ln(hT_cur + f, g2[:], b2[:])

        valid = jnp.where(pad, 0.0, 1.0).astype(jnp.float32)
        pooled_rows = []
        for bi in range(_B):
            vb = valid[bi:bi + 1, :]
            pooled_rows.append(_DOT(vb, hT_cur[bi * _T:(bi + 1) * _T, :]))
        pooled = jnp.concatenate(pooled_rows, axis=0) / nf.astype(jnp.float32)

        z = jnp.maximum(_ln(_DOT(pooled, cw0_ref[:]) + cb0_ref[:],
                            cg0_ref[:], cbt0_ref[:]), 0.0)
        z = jnp.maximum(_ln(_DOT(z, cw1_ref[:]) + cb1_ref[:],
                            cg1_ref[:], cbt1_ref[:]), 0.0)
        out_ref[:] = _DOT(z, cw2_ref[:]) + cb2_ref[:]


def kernel(x_temporal, num_frames, gcn_W0, gcn_b0, gcn_W1, gcn_b1, gcn_W2,
           gcn_b2, tin_W, tin_b, Wq0, Wk0, Wv0, Wo0, ff1_0, ff1b_0, ff2_0,
           ff2b_0, ln1g_0, ln1b_0, ln2g_0, ln2b_0, Wq1, Wk1, Wv1, Wo1, ff1_1,
           ff1b_1, ff2_1, ff2b_1, ln1g_1, ln1b_1, ln2g_1, ln2b_1, cls_W0,
           cls_b0, cls_ln0g, cls_ln0b, cls_W1, cls_b1, cls_ln1g, cls_ln1b,
           cls_W2, cls_b2):
    Bb, Tt, Nn, Ff = x_temporal.shape
    xf = x_temporal.reshape(Bb * Tt * Nn, Ff)
    A = jnp.asarray(_band_adjacency())
    r2 = lambda a: a.reshape(1, -1)

    const = lambda i: (0, 0)
    xmap = lambda i: (jnp.minimum(i, _NCHUNK - 1), 0)
    spatial_args = (xf, gcn_W0, r2(gcn_b0), gcn_W1, r2(gcn_b1), gcn_W2,
                    r2(gcn_b2), A)
    temporal_args = (
        num_frames, tin_W, r2(tin_b),
        Wq0, Wk0, Wv0, Wo0, ff1_0, r2(ff1b_0), ff2_0, r2(ff2b_0),
        r2(ln1g_0), r2(ln1b_0), r2(ln2g_0), r2(ln2b_0),
        Wq1, Wk1, Wv1, Wo1, ff1_1, r2(ff1b_1), ff2_1, r2(ff2b_1),
        r2(ln1g_1), r2(ln1b_1), r2(ln2g_1), r2(ln2b_1),
        cls_W0, r2(cls_b0), r2(cls_ln0g), r2(cls_ln0b),
        cls_W1, r2(cls_b1), r2(cls_ln1g), r2(cls_ln1b),
        cls_W2, r2(cls_b2))
    out = pl.pallas_call(
        _fused_kernel,
        grid=(_NCHUNK + 1,),
        in_specs=[
            pl.BlockSpec((_ROWS, _F), xmap),
            pl.BlockSpec((_F, _H), const),
            pl.BlockSpec((1, _H), const),
            pl.BlockSpec((_H, _H), const),
            pl.BlockSpec((1, _H), const),
            pl.BlockSpec((_H, _H), const),
            pl.BlockSpec((1, _H), const),
            pl.BlockSpec((_N, _N), const),
        ] + [pl.BlockSpec(t.shape, const) for t in temporal_args],
        out_specs=pl.BlockSpec((_B, 8), const),
        out_shape=jax.ShapeDtypeStruct((_B, 8), jnp.float32),
        scratch_shapes=[pltpu.VMEM((_G, 2 * _H), jnp.float32)],
        compiler_params=pltpu.CompilerParams(
            dimension_semantics=("arbitrary",)),
    )(*spatial_args, *temporal_args)
    return out


# speedup vs baseline: 1.1928x; 1.0524x over previous
import functools

import numpy as np
import jax
import jax.numpy as jnp
from jax.experimental import pallas as pl
from jax.experimental.pallas import tpu as pltpu

_N = 96
_K = 5
_GB = 64
_ROWS = _GB * _N
_G = 256
_F = 256
_H = 128
_HT = 256
_NH = 8
_DH = 32
_B = 8
_T = 32
_NCHUNK = _G // _GB


def _band_adjacency() -> np.ndarray:
    src, dst = [], []
    for i in range(_N):
        for j in range(max(0, i - _K), min(_N, i + _K + 1)):
            if i != j:
                src.append(i)
                dst.append(j)
    src = np.asarray(src)
    dst = np.asarray(dst)
    deg = np.zeros(_N, np.float64)
    np.add.at(deg, dst, 1.0)
    deg += 1.0
    dinv = 1.0 / np.sqrt(deg)
    A = np.zeros((_N, _N), np.float64)
    A[dst, src] = dinv[src] * dinv[dst]
    A[np.arange(_N), np.arange(_N)] = dinv * dinv
    return A.astype(np.float32)


_DOT = functools.partial(jnp.dot, preferred_element_type=jnp.float32,
                         precision=jax.lax.Precision.DEFAULT)


def _ln(x, g, b):
    m = jnp.mean(x, axis=-1, keepdims=True)
    v = jnp.mean((x - m) ** 2, axis=-1, keepdims=True)
    return (x - m) * jax.lax.rsqrt(v + 1e-5) * g + b


def _fused_kernel(x_ref, w0_ref, b0_ref, w1_ref, b1_ref, w2_ref, b2_ref,
                  a_ref, nf_ref, tinw_ref, tinb_ref,
                  wq0_ref, wk0_ref, wv0_ref, wo0_ref, f10_ref, f1b0_ref,
                  f20_ref, f2b0_ref, g10_ref, b10_ref, g20_ref, b20_ref,
                  wq1_ref, wk1_ref, wv1_ref, wo1_ref, f11_ref, f1b1_ref,
                  f21_ref, f2b1_ref, g11_ref, b11_ref, g21_ref, b21_ref,
                  cw0_ref, cb0_ref, cg0_ref, cbt0_ref,
                  cw1_ref, cb1_ref, cg1_ref, cbt1_ref,
                  cw2_ref, cb2_ref, out_ref, ge_scr):
    i = pl.program_id(0)

    @pl.when(i < _NCHUNK)
    def _spatial():
        ahat = a_ref[:]

        def band(h):
            htile = jnp.concatenate(
                [h[g * _N:(g + 1) * _N, :] for g in range(_GB)], axis=1)
            otile = _DOT(ahat, htile)
            return jnp.concatenate(
                [otile[:, g * _H:(g + 1) * _H] for g in range(_GB)], axis=0)

        h = _DOT(x_ref[:], w0_ref[:])
        h = jnp.maximum(band(h) + b0_ref[:], 0.0)
        h = _DOT(h, w1_ref[:])
        h = jnp.maximum(band(h) + b1_ref[:], 0.0)
        h = _DOT(h, w2_ref[:])
        h = band(h) + b2_ref[:]
        means, maxs = [], []
        for g in range(_GB):
            seg = h[g * _N:(g + 1) * _N, :]
            means.append(jnp.mean(seg, axis=0, keepdims=True))
            maxs.append(jnp.max(seg, axis=0, keepdims=True))
        ge_chunk = jnp.concatenate(
            [jnp.concatenate(means, axis=0), jnp.concatenate(maxs, axis=0)],
            axis=1)
        ge_scr[pl.ds(i * _GB, _GB), :] = ge_chunk

    @pl.when(i == _NCHUNK)
    def _temporal():
        hT = _DOT(ge_scr[:], tinw_ref[:]) + tinb_ref[:]
        nf = nf_ref[:]
        t_iota = jax.lax.broadcasted_iota(jnp.int32, (_B, _T), 1)
        pad = t_iota >= nf
        scale = jnp.float32(1.0 / np.sqrt(_DH))
        BT = _B * _T

        r_blk = jax.lax.broadcasted_iota(jnp.int32, (BT, BT), 0) // _DH
        c_blk = jax.lax.broadcasted_iota(jnp.int32, (BT, BT), 1) // _DH
        M = (r_blk == c_blk).astype(jnp.float32)

        mrows = []
        for bi in range(_B):
            row = jnp.where(pad[bi:bi + 1, :], jnp.float32(-1e9), 0.0)
            rowt = jnp.concatenate([row] * _NH, axis=1)
            mrows.append(jnp.broadcast_to(rowt, (_T, BT)))
        addend = jnp.concatenate(mrows, axis=0)

        layers = (
            (wq0_ref, wk0_ref, wv0_ref, wo0_ref, f10_ref, f1b0_ref, f20_ref,
             f2b0_ref, g10_ref, b10_ref, g20_ref, b20_ref),
            (wq1_ref, wk1_ref, wv1_ref, wo1_ref, f11_ref, f1b1_ref, f21_ref,
             f2b1_ref, g11_ref, b11_ref, g21_ref, b21_ref),
        )
        hT_cur = hT
        for (wq, wk, wv, wo, f1, f1b, f2, f2b, g1, b1, g2, b2) in layers:
            q = _DOT(hT_cur, wq[:])
            k = _DOT(hT_cur, wk[:])
            v = _DOT(hT_cur, wv[:])
            kT = k.T
            srows = []
            for bi in range(_B):
                kTb = kT[:, bi * _T:(bi + 1) * _T]
                Kp = jnp.concatenate([kTb] * _NH, axis=1) * M
                srows.append(_DOT(q[bi * _T:(bi + 1) * _T, :], Kp))
            S2 = jnp.concatenate(srows, axis=0) * scale + addend
            mx = jnp.max(S2)
            e = jnp.exp(S2 - mx)
            denom = _DOT(e, M)
            A2 = e / denom
            orows = []
            for bi in range(_B):
                vb = v[bi * _T:(bi + 1) * _T, :]
                Vp = jnp.concatenate([vb] * _NH, axis=0) * M
                orows.append(_DOT(A2[bi * _T:(bi + 1) * _T, :], Vp))
            O = jnp.concatenate(orows, axis=0)
            hT_cur = _ln(hT_cur + _DOT(O, wo[:]), g1[:], b1[:])
            f = _DOT(jnp.maximum(_DOT(hT_cur, f1[:]) + f1b[:], 0.0),
                     f2[:]) + f2b[:]
            hT_cur = _ln(hT_cur + f, g2[:], b2[:])

        valid = jnp.where(pad, 0.0, 1.0).astype(jnp.float32)
        pooled_rows = []
        for bi in range(_B):
            vb = valid[bi:bi + 1, :]
            pooled_rows.append(_DOT(vb, hT_cur[bi * _T:(bi + 1) * _T, :]))
        pooled = jnp.concatenate(pooled_rows, axis=0) / nf.astype(jnp.float32)

        z = jnp.maximum(_ln(_DOT(pooled, cw0_ref[:]) + cb0_ref[:],
                            cg0_ref[:], cbt0_ref[:]), 0.0)
        z = jnp.maximum(_ln(_DOT(z, cw1_ref[:]) + cb1_ref[:],
                            cg1_ref[:], cbt1_ref[:]), 0.0)
        out_ref[:] = _DOT(z, cw2_ref[:]) + cb2_ref[:]


def kernel(x_temporal, num_frames, gcn_W0, gcn_b0, gcn_W1, gcn_b1, gcn_W2,
           gcn_b2, tin_W, tin_b, Wq0, Wk0, Wv0, Wo0, ff1_0, ff1b_0, ff2_0,
           ff2b_0, ln1g_0, ln1b_0, ln2g_0, ln2b_0, Wq1, Wk1, Wv1, Wo1, ff1_1,
           ff1b_1, ff2_1, ff2b_1, ln1g_1, ln1b_1, ln2g_1, ln2b_1, cls_W0,
           cls_b0, cls_ln0g, cls_ln0b, cls_W1, cls_b1, cls_ln1g, cls_ln1b,
           cls_W2, cls_b2):
    Bb, Tt, Nn, Ff = x_temporal.shape
    xf = x_temporal.reshape(Bb * Tt * Nn, Ff)
    A = jnp.asarray(_band_adjacency())
    r2 = lambda a: a.reshape(1, -1)

    const = lambda i: (0, 0)
    xmap = lambda i: (jnp.minimum(i, _NCHUNK - 1), 0)
    spatial_args = (xf, gcn_W0, r2(gcn_b0), gcn_W1, r2(gcn_b1), gcn_W2,
                    r2(gcn_b2), A)
    temporal_args = (
        num_frames, tin_W, r2(tin_b),
        Wq0, Wk0, Wv0, Wo0, ff1_0, r2(ff1b_0), ff2_0, r2(ff2b_0),
        r2(ln1g_0), r2(ln1b_0), r2(ln2g_0), r2(ln2b_0),
        Wq1, Wk1, Wv1, Wo1, ff1_1, r2(ff1b_1), ff2_1, r2(ff2b_1),
        r2(ln1g_1), r2(ln1b_1), r2(ln2g_1), r2(ln2b_1),
        cls_W0, r2(cls_b0), r2(cls_ln0g), r2(cls_ln0b),
        cls_W1, r2(cls_b1), r2(cls_ln1g), r2(cls_ln1b),
        cls_W2, r2(cls_b2))
    out = pl.pallas_call(
        _fused_kernel,
        grid=(_NCHUNK + 1,),
        in_specs=[
            pl.BlockSpec((_ROWS, _F), xmap),
            pl.BlockSpec((_F, _H), const),
            pl.BlockSpec((1, _H), const),
            pl.BlockSpec((_H, _H), const),
            pl.BlockSpec((1, _H), const),
            pl.BlockSpec((_H, _H), const),
            pl.BlockSpec((1, _H), const),
            pl.BlockSpec((_N, _N), const),
        ] + [pl.BlockSpec(t.shape, const) for t in temporal_args],
        out_specs=pl.BlockSpec((_B, 8), const),
        out_shape=jax.ShapeDtypeStruct((_B, 8), jnp.float32),
        scratch_shapes=[pltpu.VMEM((_G, 2 * _H), jnp.float32)],
        compiler_params=pltpu.CompilerParams(
            dimension_semantics=("arbitrary",)),
    )(*spatial_args, *temporal_args)
    return out
